# baseline (speedup 1.0000x reference)
import functools

import jax
import jax.numpy as jnp
from jax import lax
from jax.experimental import pallas as pl
from jax.experimental.pallas import tpu as pltpu

_N_VPN = 64
_N_KC = 32768
_K_TOP = 1638
_ROWS_PER_BLOCK = 32
_HI_BITS = 0x3F800000
_N_ITERS = 30


def _mb_body(k_top, hi_bits, n_iters, x_ref, wt_ref, wout_ref, mask_ref,
             mbon_ref, bits_ref):
    logits = jnp.dot(x_ref[...], wt_ref[...],
                     preferred_element_type=jnp.float32)
    bits_ref[...] = lax.bitcast_convert_type(logits, jnp.int32)
    rows = x_ref.shape[0]
    n_kc_i = bits_ref.shape[1]
    k_f = jnp.float32(k_top)
    ones_c = jnp.ones((n_kc_i, 1), jnp.bfloat16)

    def count(ind_bf16):
        return jnp.dot(ind_bf16, ones_c, preferred_element_type=jnp.float32)

    lo0 = jnp.full((rows, 1), -1, jnp.int32)
    hi0 = jnp.full((rows, 1), hi_bits, jnp.int32)

    def step(_, carry):
        lo, hi = carry
        mid = lax.shift_right_arithmetic(lo + hi, 1)
        cnt = count((bits_ref[...] > mid).astype(jnp.bfloat16))
        pred = cnt < k_f
        return jnp.where(pred, lo, mid), jnp.where(pred, mid, hi)

    _, thr = lax.fori_loop(0, n_iters, step, (lo0, hi0))

    bits = bits_ref[...]
    gt = bits > thr
    eq = bits == thr
    g = count(gt.astype(jnp.bfloat16))
    r = k_f - g
    n_kc = bits.shape[1]
    col = lax.broadcasted_iota(jnp.int32, bits.shape, 1)
    clo0 = jnp.full((rows, 1), -1, jnp.int32)
    chi0 = jnp.full((rows, 1), n_kc - 1, jnp.int32)

    def cstep(_, carry):
        lo, hi = carry
        mid = lax.shift_right_arithmetic(lo + hi, 1)
        cnt = count((eq & (col <= mid)).astype(jnp.bfloat16))
        pred = cnt >= r
        return jnp.where(pred, lo, mid), jnp.where(pred, mid, hi)

    _, cthr = lax.fori_loop(0, 15, cstep, (clo0, chi0))
    maskf = (gt | (eq & (col <= cthr))).astype(jnp.float32)
    mask_ref[...] = maskf
    mbon_ref[...] = jnp.sum(maskf * wout_ref[...], axis=1, keepdims=True)


def _build(n_vpn, n_kc, k_top, rows_per_block, batch, hi_bits, n_iters,
           interpret=False):
    grid = batch // rows_per_block
    return pl.pallas_call(
        functools.partial(_mb_body, k_top, hi_bits, n_iters),
        grid=(grid,),
        in_specs=[
            pl.BlockSpec((rows_per_block, n_vpn), lambda i: (i, 0)),
            pl.BlockSpec((n_vpn, n_kc), lambda i: (0, 0)),
            pl.BlockSpec((1, n_kc), lambda i: (0, 0)),
        ],
        out_specs=[
            pl.BlockSpec((rows_per_block, n_kc), lambda i: (i, 0)),
            pl.BlockSpec((rows_per_block, 1), lambda i: (i, 0)),
        ],
        out_shape=[
            jax.ShapeDtypeStruct((batch, n_kc), jnp.float32),
            jax.ShapeDtypeStruct((batch, 1), jnp.float32),
        ],
        scratch_shapes=[pltpu.VMEM((rows_per_block, n_kc), jnp.int32)],
        interpret=interpret,
    )


def kernel(x, W, W_out):
    batch = x.shape[0]
    wt = W.T
    mask, mbon = _build(_N_VPN, _N_KC, _K_TOP, _ROWS_PER_BLOCK, batch,
                        _HI_BITS, _N_ITERS)(x, wt, W_out)
    return (mask, mbon)


# speedup vs baseline: 1.9195x; 1.9195x over previous
import functools

import jax
import jax.numpy as jnp
from jax import lax
from jax.experimental import pallas as pl
from jax.experimental.pallas import tpu as pltpu

_N_VPN = 64
_N_KC = 32768
_K_TOP = 1638
_ROWS_PER_BLOCK = 32
_HI_BITS = 0x3F800000


def _mb_body(k_top, hi_bits, x_ref, wt_ref, wout_t_ref, mask_ref, mbon_ref,
             bits_ref):
    rows = x_ref.shape[0]
    n_kc = bits_ref.shape[1]
    logits = jnp.dot(x_ref[...], wt_ref[...],
                     preferred_element_type=jnp.float32)
    bits_ref[...] = lax.bitcast_convert_type(logits, jnp.int32)

    k_f = jnp.float32(k_top)
    def count(ind_bool):
        return jnp.sum(ind_bool.astype(jnp.float32), axis=1, keepdims=True)

    lo0 = jnp.full((rows, 1), -1, jnp.int32)
    hi0 = jnp.full((rows, 1), hi_bits, jnp.int32)
    cl0 = jnp.full((rows, 1), float(n_kc), jnp.float32)
    ch0 = jnp.zeros((rows, 1), jnp.float32)
    f10 = jnp.ones((rows, 1), jnp.float32)
    first_probe = lax.bitcast_convert_type(jnp.float32(0.66), jnp.int32)

    def vdone(lo, hi, ch):
        return (hi - lo <= 1) | (ch == k_f)

    def probe(i, lo, hi, cl, ch, fl, fh, done):
        v_lo = lax.bitcast_convert_type(jnp.maximum(lo, 0), jnp.float32)
        v_hi = lax.bitcast_convert_type(hi, jnp.float32)
        ecl = (cl - k_f) * fl
        ech = (k_f - ch) * fh
        frac = ecl / jnp.maximum(ecl + ech, jnp.float32(1e-9))
        interp = lax.bitcast_convert_type(v_lo + (v_hi - v_lo) * frac,
                                          jnp.int32)
        bis = lax.shift_right_arithmetic(lo + hi, 1)
        cand = jnp.where((i & 3) == 3, bis, interp)
        cand = jnp.where(i == 0, jnp.full_like(cand, first_probe), cand)
        mid = jnp.clip(cand, lo + 1, hi - 1)
        return jnp.where(done, lo, mid)

    def vcond(st):
        _, lo, hi, _, ch, _, _ = st
        return jnp.any(~vdone(lo, hi, ch))

    def vbody(st):
        i, lo, hi, cl, ch, fl, fh = st
        done = vdone(lo, hi, ch)
        mid = probe(i, lo, hi, cl, ch, fl, fh, done)
        bits = bits_ref[...]
        gt = bits > mid
        cnt = count(gt)
        ms = jnp.max(jnp.where(gt, -1, bits), axis=1, keepdims=True)
        ml = jnp.min(jnp.where(gt, bits, hi_bits), axis=1, keepdims=True)
        pred = cnt < k_f
        upd = lambda a, b: jnp.where(done, a, jnp.where(pred, a, b))
        upd2 = lambda a, b: jnp.where(done, a, jnp.where(pred, b, a))
        return (i + 1, upd(lo, ml - 1), upd2(hi, ms), upd(cl, cnt),
                upd2(ch, cnt),
                jnp.where(done, fl, jnp.where(pred, fl * 0.5, 1.0)),
                jnp.where(done, fh, jnp.where(pred, 1.0, fh * 0.5)))

    _, lo, thr, cl, ch, _, _ = lax.while_loop(
        vcond, vbody, (0, lo0, hi0, cl0, ch0, f10, f10))
    r = k_f - ch
    eq_tot = cl - ch

    bits = bits_ref[...]
    eq = bits == thr
    col = lax.broadcasted_iota(jnp.int32, (rows, n_kc), 1)

    clo0 = jnp.full((rows, 1), -1, jnp.int32)
    chi0 = jnp.full((rows, 1), n_kc - 1, jnp.int32)
    dl0 = jnp.zeros((rows, 1), jnp.float32)

    def cdone(clo, chi, dh):
        return (chi - clo <= 1) | (dh == r) | (r <= 0)

    def ccond(st):
        _, clo, chi, _, dh = st
        return jnp.any(~cdone(clo, chi, dh))

    def cbody(st):
        i, clo, chi, dl, dh = st
        frac = (r - dl) / jnp.maximum(dh - dl, jnp.float32(1e-9))
        interp = clo + ((chi - clo).astype(jnp.float32) *
                        frac).astype(jnp.int32)
        bis = lax.shift_right_arithmetic(clo + chi, 1)
        cand = jnp.where((i & 3) == 3, bis, interp)
        mid = jnp.clip(cand, clo + 1, chi - 1)
        done = cdone(clo, chi, dh)
        mid = jnp.where(done, clo, mid)
        inwin = eq & (col <= mid)
        cnt = count(inwin)
        ms = jnp.max(jnp.where(inwin, col, -1), axis=1, keepdims=True)
        ml = jnp.min(jnp.where(eq & (col > mid), col, n_kc), axis=1,
                     keepdims=True)
        pred = cnt < r
        upd = lambda a, b: jnp.where(done, a, jnp.where(pred, a, b))
        upd2 = lambda a, b: jnp.where(done, a, jnp.where(pred, b, a))
        return (i + 1, upd2(clo, ml - 1), upd(chi, ms), upd2(dl, cnt),
                upd(dh, cnt))

    _, _, chi, _, _ = lax.while_loop(ccond, cbody,
                                     (0, clo0, chi0, dl0, eq_tot))
    cstar = jnp.where(r <= 0, jnp.full_like(chi, -1), chi)

    maskf = ((bits > thr) | (eq & (col <= cstar))).astype(jnp.float32)
    mask_ref[...] = maskf
    mbon_ref[...] = jnp.dot(maskf, wout_t_ref[...],
                            preferred_element_type=jnp.float32)


def _build(n_vpn, n_kc, k_top, rows_per_block, batch, hi_bits,
           interpret=False):
    grid = batch // rows_per_block
    return pl.pallas_call(
        functools.partial(_mb_body, k_top, hi_bits),
        grid=(grid,),
        in_specs=[
            pl.BlockSpec((rows_per_block, n_vpn), lambda i: (i, 0)),
            pl.BlockSpec((n_vpn, n_kc), lambda i: (0, 0)),
            pl.BlockSpec((n_kc, 1), lambda i: (0, 0)),
        ],
        out_specs=[
            pl.BlockSpec((rows_per_block, n_kc), lambda i: (i, 0)),
            pl.BlockSpec((rows_per_block, 1), lambda i: (i, 0)),
        ],
        out_shape=[
            jax.ShapeDtypeStruct((batch, n_kc), jnp.float32),
            jax.ShapeDtypeStruct((batch, 1), jnp.float32),
        ],
        scratch_shapes=[pltpu.VMEM((rows_per_block, n_kc), jnp.int32)],
        interpret=interpret,
    )


def kernel(x, W, W_out):
    batch = x.shape[0]
    mask, mbon = _build(_N_VPN, _N_KC, _K_TOP, _ROWS_PER_BLOCK, batch,
                        _HI_BITS)(x, W.T, W_out.T)
    return (mask, mbon)
